# baseline (speedup 1.0000x reference)
import jax
import jax.numpy as jnp
from jax.experimental import pallas as pl

_NUM_HEADS = 16
_MAX_DIST = 12
_NB = _MAX_DIST + 1
_V = 1024
_ROWS_PER_STEP = 256
_GROUPS = _ROWS_PER_STEP // 8
_GRID = _V // _ROWS_PER_STEP


def _bias_kernel(dist_ref, lhs_ref, out_ref):
    lhs = lhs_ref[...]
    for g in range(_GROUPS):
        tile = jnp.clip(dist_ref[g * 8:(g + 1) * 8, :], 0, _MAX_DIST)
        iota = jax.lax.broadcasted_iota(jnp.int32, (_NB, 8, _V), 0)
        oh = (tile[None] == iota).astype(jnp.float32)
        rhs = oh.reshape(_NB * 8, _V)
        res = jax.lax.dot(lhs, rhs, preferred_element_type=jnp.float32)
        out_ref[:, g * 8:(g + 1) * 8, :] = res.reshape(_NUM_HEADS, 8, _V)


def kernel(dist_matrix, bias_table):
    eye8 = jnp.eye(8, dtype=jnp.float32)
    lhs = (bias_table[:, None, :, None] * eye8[None, :, None, :])
    lhs = lhs.reshape(_NUM_HEADS * 8, _NB * 8)
    return pl.pallas_call(
        _bias_kernel,
        grid=(_GRID,),
        in_specs=[
            pl.BlockSpec((_ROWS_PER_STEP, _V), lambda i: (i, 0)),
            pl.BlockSpec((_NUM_HEADS * 8, _NB * 8), lambda i: (0, 0)),
        ],
        out_specs=pl.BlockSpec(
            (_NUM_HEADS, _ROWS_PER_STEP, _V), lambda i: (0, i, 0)),
        out_shape=jax.ShapeDtypeStruct((_NUM_HEADS, _V, _V), jnp.float32),
    )(dist_matrix.astype(jnp.int32), lhs)


# speedup vs baseline: 1.0734x; 1.0734x over previous
import jax
import jax.numpy as jnp
from jax.experimental import pallas as pl

_NUM_HEADS = 16
_MAX_DIST = 12
_NB = _MAX_DIST + 1
_V = 1024
_ROWS_PER_STEP = 128
_GROUPS = _ROWS_PER_STEP // 8
_GRID = _V // _ROWS_PER_STEP


def _bias_kernel(dist_ref, lhs_ref, out_ref):
    lhs = lhs_ref[...]
    for g in range(_GROUPS):
        tile = jnp.clip(dist_ref[g * 8:(g + 1) * 8, :], 0, _MAX_DIST)
        iota = jax.lax.broadcasted_iota(jnp.int32, (_NB, 8, _V), 0)
        oh = (tile[None] == iota).astype(jnp.bfloat16)
        rhs = oh.reshape(_NB * 8, _V)
        res = jax.lax.dot(lhs, rhs, preferred_element_type=jnp.float32)
        out_ref[:, g * 8:(g + 1) * 8, :] = res.reshape(_NUM_HEADS, 8, _V)


def kernel(dist_matrix, bias_table):
    eye8 = jnp.eye(8, dtype=jnp.float32)
    lhs = (bias_table[:, None, :, None] * eye8[None, :, None, :])
    lhs = lhs.reshape(_NUM_HEADS * 8, _NB * 8).astype(jnp.bfloat16)
    return pl.pallas_call(
        _bias_kernel,
        grid=(_GRID,),
        in_specs=[
            pl.BlockSpec((_ROWS_PER_STEP, _V), lambda i: (i, 0)),
            pl.BlockSpec((_NUM_HEADS * 8, _NB * 8), lambda i: (0, 0)),
        ],
        out_specs=pl.BlockSpec(
            (_NUM_HEADS, _ROWS_PER_STEP, _V), lambda i: (0, i, 0)),
        out_shape=jax.ShapeDtypeStruct((_NUM_HEADS, _V, _V), jnp.float32),
    )(dist_matrix.astype(jnp.int32), lhs)
